# Initial kernel scaffold; baseline (speedup 1.0000x reference)
import jax
import jax.numpy as jnp
from jax.experimental import pallas as pl


def kernel(query, mem_keys, mem_values, W_q, b_q, k):
    raise NotImplementedError("write your pallas kernel here")


# speedup vs baseline: 1.6319x; 1.6319x over previous
import functools

import jax
import jax.numpy as jnp
from jax import lax
from jax.experimental import pallas as pl
from jax.experimental.pallas import tpu as pltpu
from jax.experimental.pallas import tpu_sc as plsc

B = 1024
C = 100000
KD = 128
VD = 128
K = 16
CB = 1024
NB = (C + CB - 1) // CB

_NEG_INF = float("-inf")
_I32_MAX = jnp.iinfo(jnp.int32).max


def _topk_body(query_ref, wq_ref, bq_ref, keys_ref, att_ref, idx_ref,
               q_s, rv, ri):
    b = pl.program_id(0)

    @pl.when(b == 0)
    def _init():
        q = lax.dot_general(query_ref[...], wq_ref[...],
                            (((1,), (1,)), ((), ())),
                            preferred_element_type=jnp.float32)
        q_s[...] = q + bq_ref[...]
        rv[...] = jnp.full((B, K), _NEG_INF, dtype=jnp.float32)
        ri[...] = jnp.zeros((B, K), dtype=jnp.int32)

    s = lax.dot_general(q_s[...], keys_ref[...],
                        (((1,), (1,)), ((), ())),
                        preferred_element_type=jnp.float32)
    col = b * CB + lax.broadcasted_iota(jnp.int32, (B, CB), 1)
    s = jnp.where(col < C, s, _NEG_INF)

    vals = jnp.concatenate([rv[...], s], axis=1)
    idxs = jnp.concatenate([ri[...], col], axis=1)

    out_v = []
    out_i = []
    for _ in range(K):
        m = jnp.max(vals, axis=1, keepdims=True)
        cand = jnp.where(vals == m, idxs, _I32_MAX)
        mi = jnp.min(cand, axis=1, keepdims=True)
        out_v.append(m)
        out_i.append(mi)
        vals = jnp.where((vals == m) & (idxs == mi), _NEG_INF, vals)
    rv[...] = jnp.concatenate(out_v, axis=1)
    ri[...] = jnp.concatenate(out_i, axis=1)

    @pl.when(b == NB - 1)
    def _finish():
        top = rv[...]
        e = jnp.exp(top - top[:, 0:1])
        att_ref[...] = e / jnp.sum(e, axis=1, keepdims=True)
        idx_ref[...] = ri[...]


def _score_topk(query, mem_keys, W_q, b_q):
    return pl.pallas_call(
        _topk_body,
        grid=(NB,),
        in_specs=[
            pl.BlockSpec((B, KD), lambda b: (0, 0)),
            pl.BlockSpec((KD, KD), lambda b: (0, 0)),
            pl.BlockSpec((1, KD), lambda b: (0, 0)),
            pl.BlockSpec((CB, KD), lambda b: (b, 0)),
        ],
        out_specs=[
            pl.BlockSpec((B, K), lambda b: (0, 0)),
            pl.BlockSpec((B, K), lambda b: (0, 0)),
        ],
        out_shape=[
            jax.ShapeDtypeStruct((B, K), jnp.float32),
            jax.ShapeDtypeStruct((B, K), jnp.int32),
        ],
        scratch_shapes=[
            pltpu.VMEM((B, KD), jnp.float32),
            pltpu.VMEM((B, K), jnp.float32),
            pltpu.VMEM((B, K), jnp.int32),
        ],
    )(query, W_q, b_q.reshape(1, KD), mem_keys)


_NW = 32
_BPW = (B * K) // _NW
_CHUNK = 128
_NCH = _BPW // _CHUNK


def _gather_kernel(values_hbm, idx_hbm, out_hbm, idx_v, rows_v, sem):
    wid = lax.axis_index("s") * 2 + lax.axis_index("c")
    base = wid * _BPW
    pltpu.sync_copy(idx_hbm.at[pl.ds(base, _BPW)], idx_v)
    copies = []
    for j in range(_NCH):
        copies.append(pltpu.async_copy(
            values_hbm.at[idx_v.at[pl.ds(j * _CHUNK, _CHUNK)]],
            rows_v.at[pl.ds(j * _CHUNK, _CHUNK)],
            sem,
        ))
    for cp in copies:
        cp.wait()
    pltpu.sync_copy(rows_v, out_hbm.at[pl.ds(base, _BPW)])


def _gather_values(mem_values, top_idx):
    mesh = plsc.VectorSubcoreMesh(core_axis_name="c", subcore_axis_name="s")
    call = functools.partial(
        pl.kernel,
        mesh=mesh,
        out_type=jax.ShapeDtypeStruct((B * K, VD), jnp.float32),
        scratch_types=[
            pltpu.VMEM((_BPW,), jnp.int32),
            pltpu.VMEM((_BPW, VD), jnp.float32),
            pltpu.SemaphoreType.DMA,
        ],
    )(_gather_kernel)
    flat = call(mem_values, top_idx.reshape(B * K))
    return flat.reshape(B, K, VD)


def kernel(query, mem_keys, mem_values, W_q, b_q, k):
    att, top_idx = _score_topk(query, mem_keys, W_q, b_q)
    retrieved = _gather_values(mem_values, top_idx)
    return retrieved, att


# speedup vs baseline: 2.0158x; 1.2352x over previous
import functools

import jax
import jax.numpy as jnp
from jax import lax
from jax.experimental import pallas as pl
from jax.experimental.pallas import tpu as pltpu
from jax.experimental.pallas import tpu_sc as plsc

B = 1024
C = 100000
KD = 128
VD = 128
K = 16
CB = 1024
NB = (C + CB - 1) // CB

_NEG_INF = float("-inf")
_I32_MAX = jnp.iinfo(jnp.int32).max


def _blocks_body(query_ref, wq_ref, bq_ref, keys_ref, cv_ref, ci_ref, q_s):
    b = pl.program_id(0)

    @pl.when(b == 0)
    def _init():
        q = lax.dot_general(query_ref[...], wq_ref[...],
                            (((1,), (1,)), ((), ())),
                            preferred_element_type=jnp.float32)
        q_s[...] = q + bq_ref[...]

    s = lax.dot_general(q_s[...], keys_ref[...],
                        (((1,), (1,)), ((), ())),
                        preferred_element_type=jnp.float32)
    pos = lax.broadcasted_iota(jnp.int32, (1, CB), 1)
    s = jnp.where(b * CB + pos < C, s, _NEG_INF)

    out_v = []
    out_i = []
    for _ in range(K):
        m = jnp.max(s, axis=1, keepdims=True)
        cand = jnp.where(s == m, pos, _I32_MAX)
        mi = jnp.min(cand, axis=1, keepdims=True)
        out_v.append(m)
        out_i.append(mi)
        s = jnp.where(cand == mi, _NEG_INF, s)
    cv_ref[...] = jnp.concatenate(out_v, axis=1).reshape(1, B, K)
    ci_ref[...] = (b * CB + jnp.concatenate(out_i, axis=1)).reshape(1, B, K)


def _score_blocks(query, mem_keys, W_q, b_q):
    return pl.pallas_call(
        _blocks_body,
        grid=(NB,),
        in_specs=[
            pl.BlockSpec((B, KD), lambda b: (0, 0)),
            pl.BlockSpec((KD, KD), lambda b: (0, 0)),
            pl.BlockSpec((1, KD), lambda b: (0, 0)),
            pl.BlockSpec((CB, KD), lambda b: (b, 0)),
        ],
        out_specs=[
            pl.BlockSpec((1, B, K), lambda b: (b, 0, 0)),
            pl.BlockSpec((1, B, K), lambda b: (b, 0, 0)),
        ],
        out_shape=[
            jax.ShapeDtypeStruct((NB, B, K), jnp.float32),
            jax.ShapeDtypeStruct((NB, B, K), jnp.int32),
        ],
        scratch_shapes=[
            pltpu.VMEM((B, KD), jnp.float32),
        ],
    )(query, W_q, b_q.reshape(1, KD), mem_keys)


_RB = 256
_NC = NB * K


def _merge_body(cv_ref, ci_ref, att_ref, idx_ref):
    v = cv_ref[...]
    gi = ci_ref[...]
    top_v = []
    top_i = []
    for _ in range(K):
        m = jnp.max(v, axis=1, keepdims=True)
        candi = jnp.where(v == m, gi, _I32_MAX)
        mi = jnp.min(candi, axis=1, keepdims=True)
        top_v.append(m)
        top_i.append(mi)
        v = jnp.where(candi == mi, _NEG_INF, v)
    top = jnp.concatenate(top_v, axis=1)
    e = jnp.exp(top - top[:, 0:1])
    att_ref[...] = e / jnp.sum(e, axis=1, keepdims=True)
    idx_ref[...] = jnp.concatenate(top_i, axis=1)


def _merge(cv, ci):
    return pl.pallas_call(
        _merge_body,
        grid=(B // _RB,),
        in_specs=[
            pl.BlockSpec((_RB, _NC), lambda r: (r, 0)),
            pl.BlockSpec((_RB, _NC), lambda r: (r, 0)),
        ],
        out_specs=[
            pl.BlockSpec((_RB, K), lambda r: (r, 0)),
            pl.BlockSpec((_RB, K), lambda r: (r, 0)),
        ],
        out_shape=[
            jax.ShapeDtypeStruct((B, K), jnp.float32),
            jax.ShapeDtypeStruct((B, K), jnp.int32),
        ],
    )(cv, ci)


def _score_topk(query, mem_keys, W_q, b_q):
    cand_v, cand_i = _score_blocks(query, mem_keys, W_q, b_q)
    cv = cand_v.transpose(1, 0, 2).reshape(B, _NC)
    ci = cand_i.transpose(1, 0, 2).reshape(B, _NC)
    return _merge(cv, ci)


_NW = 32
_BPW = (B * K) // _NW
_CHUNK = 128
_NCH = _BPW // _CHUNK


def _gather_kernel(values_hbm, idx_hbm, out_hbm, idx_v, rows_v, sem):
    wid = lax.axis_index("s") * 2 + lax.axis_index("c")
    base = wid * _BPW
    pltpu.sync_copy(idx_hbm.at[pl.ds(base, _BPW)], idx_v)
    copies = []
    for j in range(_NCH):
        copies.append(pltpu.async_copy(
            values_hbm.at[idx_v.at[pl.ds(j * _CHUNK, _CHUNK)]],
            rows_v.at[pl.ds(j * _CHUNK, _CHUNK)],
            sem,
        ))
    for cp in copies:
        cp.wait()
    pltpu.sync_copy(rows_v, out_hbm.at[pl.ds(base, _BPW)])


def _gather_values(mem_values, top_idx):
    mesh = plsc.VectorSubcoreMesh(core_axis_name="c", subcore_axis_name="s")
    call = functools.partial(
        pl.kernel,
        mesh=mesh,
        out_type=jax.ShapeDtypeStruct((B * K, VD), jnp.float32),
        scratch_types=[
            pltpu.VMEM((_BPW,), jnp.int32),
            pltpu.VMEM((_BPW, VD), jnp.float32),
            pltpu.SemaphoreType.DMA,
        ],
    )(_gather_kernel)
    flat = call(mem_values, top_idx.reshape(B * K))
    return flat.reshape(B, K, VD)


def kernel(query, mem_keys, mem_values, W_q, b_q, k):
    att, top_idx = _score_topk(query, mem_keys, W_q, b_q)
    retrieved = _gather_values(mem_values, top_idx)
    return retrieved, att


# speedup vs baseline: 2.6191x; 1.2993x over previous
import functools

import jax
import jax.numpy as jnp
from jax import lax
from jax.experimental import pallas as pl
from jax.experimental.pallas import tpu as pltpu
from jax.experimental.pallas import tpu_sc as plsc

B = 1024
C = 100000
KD = 128
VD = 128
K = 16
CB = 1024
NB = (C + CB - 1) // CB
G = 8
NF = CB // G
NCAND = K * G

_NEG_INF = float("-inf")
_I32_MAX = jnp.iinfo(jnp.int32).max


def _fold_body(query_ref, wq_ref, bq_ref, keys_ref, fv_ref, fp_ref, s_ref,
               q_s):
    b = pl.program_id(0)

    @pl.when(b == 0)
    def _init():
        q = lax.dot_general(query_ref[...], wq_ref[...],
                            (((1,), (1,)), ((), ())),
                            preferred_element_type=jnp.float32)
        q_s[...] = q + bq_ref[...]

    s = lax.dot_general(q_s[...], keys_ref[...],
                        (((1,), (1,)), ((), ())),
                        preferred_element_type=jnp.float32)
    pos = b * CB + lax.broadcasted_iota(jnp.int32, (1, CB), 1)
    s = jnp.where(pos < C, s, _NEG_INF)
    s_ref[...] = s.reshape(1, B, CB)

    v = s.reshape(B, G, NF)
    p = jnp.broadcast_to(pos.reshape(1, G, NF), (B, G, NF))
    for half in (4, 2, 1):
        va, vb = v[:, :half, :], v[:, half:, :]
        pa, pb = p[:, :half, :], p[:, half:, :]
        gt = (va > vb) | ((va == vb) & (pa < pb))
        v = jnp.where(gt, va, vb)
        p = jnp.where(gt, pa, pb)
    fv_ref[...] = v.reshape(1, B, NF)
    fp_ref[...] = p.reshape(1, B, NF)


def _fold_call(query, mem_keys, W_q, b_q):
    return pl.pallas_call(
        _fold_body,
        grid=(NB,),
        in_specs=[
            pl.BlockSpec((B, KD), lambda b: (0, 0)),
            pl.BlockSpec((KD, KD), lambda b: (0, 0)),
            pl.BlockSpec((1, KD), lambda b: (0, 0)),
            pl.BlockSpec((CB, KD), lambda b: (b, 0)),
        ],
        out_specs=[
            pl.BlockSpec((1, B, NF), lambda b: (b, 0, 0)),
            pl.BlockSpec((1, B, NF), lambda b: (b, 0, 0)),
            pl.BlockSpec((1, B, CB), lambda b: (b, 0, 0)),
        ],
        out_shape=[
            jax.ShapeDtypeStruct((NB, B, NF), jnp.float32),
            jax.ShapeDtypeStruct((NB, B, NF), jnp.int32),
            jax.ShapeDtypeStruct((NB, B, CB), jnp.float32),
        ],
        scratch_shapes=[
            pltpu.VMEM((B, KD), jnp.float32),
        ],
    )(query, W_q, b_q.reshape(1, KD), mem_keys)


_RBB = 64


def _gtop_body(fv_ref, fp_ref, pos_ref):
    v = fv_ref[...]
    p = fp_ref[...]
    top_p = []
    for _ in range(K):
        m = jnp.max(jnp.max(v, axis=2, keepdims=True), axis=0,
                    keepdims=True)
        cand = jnp.where(v == m, p, _I32_MAX)
        mi = jnp.min(jnp.min(cand, axis=2, keepdims=True), axis=0,
                     keepdims=True)
        top_p.append(mi.reshape(_RBB, 1))
        v = jnp.where(cand == mi, _NEG_INF, v)
    pos_ref[...] = jnp.concatenate(top_p, axis=1)


def _gtop_call(fv, fp):
    return pl.pallas_call(
        _gtop_body,
        grid=(B // _RBB,),
        in_specs=[
            pl.BlockSpec((NB, _RBB, NF), lambda r: (0, r, 0)),
            pl.BlockSpec((NB, _RBB, NF), lambda r: (0, r, 0)),
        ],
        out_specs=pl.BlockSpec((_RBB, K), lambda r: (r, 0)),
        out_shape=jax.ShapeDtypeStruct((B, K), jnp.int32),
    )(fv, fp)


_RBC = 256


def _select_body(gs_ref, ci_ref, att_ref, idx_ref):
    s = gs_ref[...]
    gi = ci_ref[...]
    top_v = []
    top_i = []
    for _ in range(K):
        m = jnp.max(s, axis=1, keepdims=True)
        cand = jnp.where(s == m, gi, _I32_MAX)
        mi = jnp.min(cand, axis=1, keepdims=True)
        top_v.append(m)
        top_i.append(mi)
        s = jnp.where(cand == mi, _NEG_INF, s)
    top = jnp.concatenate(top_v, axis=1)
    e = jnp.exp(top - top[:, 0:1])
    att_ref[...] = e / jnp.sum(e, axis=1, keepdims=True)
    idx_ref[...] = jnp.concatenate(top_i, axis=1)


def _select_call(gscores, cidx):
    return pl.pallas_call(
        _select_body,
        grid=(B // _RBC,),
        in_specs=[
            pl.BlockSpec((_RBC, NCAND), lambda r: (r, 0)),
            pl.BlockSpec((_RBC, NCAND), lambda r: (r, 0)),
        ],
        out_specs=[
            pl.BlockSpec((_RBC, K), lambda r: (r, 0)),
            pl.BlockSpec((_RBC, K), lambda r: (r, 0)),
        ],
        out_shape=[
            jax.ShapeDtypeStruct((B, K), jnp.float32),
            jax.ShapeDtypeStruct((B, K), jnp.int32),
        ],
    )(gscores, cidx)


_NW = 32


def _make_gather(out_shape, table_rank2, n_idx):
    bpw = n_idx // _NW
    chunk = 128
    nch = bpw // chunk
    wave = min(nch, 4)
    mesh = plsc.VectorSubcoreMesh(core_axis_name="c", subcore_axis_name="s")
    if table_rank2:
        buf = pltpu.VMEM((wave * chunk, out_shape[1]), jnp.float32)
    else:
        buf = pltpu.VMEM((wave * chunk,), jnp.float32)

    def body(table_hbm, idx_hbm, out_hbm, idx_v, rows_v, sem):
        wid = lax.axis_index("s") * 2 + lax.axis_index("c")
        base = wid * bpw
        pltpu.sync_copy(idx_hbm.at[pl.ds(base, bpw)], idx_v)

        def do_wave(w):
            off = w * wave * chunk
            copies = []
            for j in range(wave):
                copies.append(pltpu.async_copy(
                    table_hbm.at[idx_v.at[pl.ds(off + j * chunk, chunk)]],
                    rows_v.at[pl.ds(j * chunk, chunk)],
                    sem,
                ))
            for cp in copies:
                cp.wait()
            pltpu.sync_copy(rows_v, out_hbm.at[pl.ds(base + off, wave * chunk)])

        if nch == wave:
            do_wave(0)
        else:
            pl.loop(0, nch // wave)(do_wave)

    call = functools.partial(
        pl.kernel,
        mesh=mesh,
        out_type=jax.ShapeDtypeStruct(out_shape, jnp.float32),
        scratch_types=[
            pltpu.VMEM((bpw,), jnp.int32),
            buf,
            pltpu.SemaphoreType.DMA,
        ],
    )(body)
    return call


def kernel(query, mem_keys, mem_values, W_q, b_q, k):
    fv, fp, scores = _fold_call(query, mem_keys, W_q, b_q)
    gpos = _gtop_call(fv, fp)
    lane = gpos % NF
    blk = gpos // CB
    members = (blk * CB + lane)[:, :, None] + NF * jnp.arange(G, dtype=jnp.int32)
    members = jnp.minimum(members, C - 1)
    cidx = members.reshape(B, NCAND)
    rows = jnp.arange(B, dtype=jnp.int32)[:, None]
    sflat = (cidx // CB) * (B * CB) + rows * CB + cidx % CB
    gscores = _make_gather((B * NCAND,), False, B * NCAND)(
        scores.reshape(NB * B * CB), sflat.reshape(B * NCAND)
    ).reshape(B, NCAND)
    att, top_idx = _select_call(gscores, cidx)
    retrieved = _make_gather((B * K, VD), True, B * K)(
        mem_values, top_idx.reshape(B * K)).reshape(B, K, VD)
    return retrieved, att
